# Initial kernel scaffold; baseline (speedup 1.0000x reference)
import jax
import jax.numpy as jnp
from jax.experimental import pallas as pl


def kernel(x, edge_index, Wfc, bfc, Wg1, bg1, Wg2, bg2, Wg3, bg3, Wih1, Whh1, bih1, bhh1, Wih2, Whh2, bih2, bhh2, Wih3, Whh3, bih3, bhh3):
    raise NotImplementedError("write your pallas kernel here")


# speedup vs baseline: 9.6408x; 9.6408x over previous
import functools

import jax
import jax.numpy as jnp
from jax import lax
from jax.experimental import pallas as pl
from jax.experimental.pallas import tpu as pltpu
from jax.experimental.pallas import tpu_sc as plsc

_NC = 2
_NS = 16
_NW = _NC * _NS
_CH = 128
_R = 256


def _sc_mesh():
    return plsc.VectorSubcoreMesh(core_axis_name="c", subcore_axis_name="s",
                                  num_cores=_NC, num_subcores=_NS)


def _make_deg_kernel(n_pad, cpt):
    rpt = n_pad // _NS

    @functools.partial(
        pl.kernel,
        out_type=jax.ShapeDtypeStruct((_NC, n_pad, 16), jnp.float32),
        mesh=_sc_mesh(),
        scratch_types=[
            pltpu.VMEM((cpt, _CH), jnp.int32),
            pltpu.VMEM((_CH, 16), jnp.float32),
            pltpu.VMEM_SHARED((n_pad, 16), jnp.float32),
            pltpu.SemaphoreType.DMA,
        ],
    )
    def deg_kernel(row_hbm, ones_hbm, zrow_hbm, out_hbm, rowv, onev, shared, sem):
        cid = lax.axis_index("c")
        sid = lax.axis_index("s")
        wid = sid * _NC + cid
        pltpu.sync_copy(row_hbm.at[wid], rowv)
        pltpu.sync_copy(ones_hbm, onev)
        pltpu.sync_copy(zrow_hbm, shared.at[pl.ds(sid * rpt, rpt)])
        plsc.subcore_barrier()

        def body(j, carry):
            pltpu.sync_copy(onev, shared.at[rowv.at[j]], add=True)
            return carry

        lax.fori_loop(0, cpt, body, 0)
        plsc.subcore_barrier()
        pltpu.sync_copy(shared.at[pl.ds(sid * rpt, rpt)],
                        out_hbm.at[cid, pl.ds(sid * rpt, rpt)])

    return deg_kernel


def _make_spmm_kernel(n_pad, cpt, d):
    rpt = n_pad // _NS

    @functools.partial(
        pl.kernel,
        out_type=jax.ShapeDtypeStruct((_NC, n_pad, d), jnp.float32),
        mesh=_sc_mesh(),
        scratch_types=[
            pltpu.VMEM((cpt, _CH), jnp.int32),
            pltpu.VMEM((cpt, _CH), jnp.int32),
            pltpu.VMEM((_CH, d), jnp.float32),
            pltpu.VMEM_SHARED((n_pad, d), jnp.float32),
            pltpu.SemaphoreType.DMA,
        ],
    )
    def spmm(table_hbm, row_hbm, col_hbm, zrow_hbm, out_hbm,
             rowv, colv, gbuf, shared, sem):
        cid = lax.axis_index("c")
        sid = lax.axis_index("s")
        wid = sid * _NC + cid
        pltpu.sync_copy(row_hbm.at[wid], rowv)
        pltpu.sync_copy(col_hbm.at[wid], colv)
        pltpu.sync_copy(zrow_hbm, shared.at[pl.ds(sid * rpt, rpt)])
        plsc.subcore_barrier()

        def body(j, carry):
            pltpu.async_copy(table_hbm.at[rowv.at[j]], gbuf, sem).wait()
            pltpu.sync_copy(gbuf, shared.at[colv.at[j]], add=True)
            return carry

        lax.fori_loop(0, cpt, body, 0)
        plsc.subcore_barrier()
        pltpu.sync_copy(shared.at[pl.ds(sid * rpt, rpt)],
                        out_hbm.at[cid, pl.ds(sid * rpt, rpt)])

    return spmm


def _leaky(v):
    return jnp.where(v >= 0, v, 0.01 * v)


def _dinv_of(d0, d1):
    return lax.rsqrt(d0[:, 0:1] + d1[:, 0:1] + 1.0)


def _make_prep_kernel(n_pad, d):
    def body(x_ref, wfc_ref, bfc_ref, d0_ref, d1_ref, wg_ref, bg_ref,
             h_ref, s_ref):
        dinv = _dinv_of(d0_ref[...], d1_ref[...])
        h = jnp.dot(x_ref[...], wfc_ref[...],
                    preferred_element_type=jnp.float32) + bfc_ref[...]
        h = _leaky(h)
        s = dinv * (jnp.dot(h, wg_ref[...],
                            preferred_element_type=jnp.float32) + bg_ref[...])
        h_ref[...] = h
        s_ref[...] = s

    row_blk = pl.BlockSpec((_R, d), lambda i: (i, 0))
    return pl.pallas_call(
        body,
        grid=(n_pad // _R,),
        in_specs=[
            row_blk,
            pl.BlockSpec((d, d), lambda i: (0, 0)),
            pl.BlockSpec((1, d), lambda i: (0, 0)),
            pl.BlockSpec((_R, 16), lambda i: (i, 0)),
            pl.BlockSpec((_R, 16), lambda i: (i, 0)),
            pl.BlockSpec((d, d), lambda i: (0, 0)),
            pl.BlockSpec((1, d), lambda i: (0, 0)),
        ],
        out_specs=[row_blk, row_blk],
        out_shape=[jax.ShapeDtypeStruct((n_pad, d), jnp.float32)] * 2,
    )


def _make_layer_kernel(n_pad, d, first, last):
    def body(*refs):
        it = iter(refs)
        p0, p1, s, h = (next(it) for _ in range(4))
        m = None if first else next(it)
        d0, d1, wih, whh, bih, bhh = (next(it) for _ in range(6))
        if not last:
            wg, bg = next(it), next(it)
        h_out = next(it)
        m_out = next(it)
        if not last:
            s_out = next(it)

        dinv = _dinv_of(d0[...], d1[...])
        g = dinv * (p0[...] + p1[...] + s[...])
        hh = h[...]
        gi = jnp.dot(g, wih[...], preferred_element_type=jnp.float32) + bih[...]
        gh = jnp.dot(hh, whh[...], preferred_element_type=jnp.float32) + bhh[...]
        r = jax.nn.sigmoid(gi[:, :d] + gh[:, :d])
        z = jax.nn.sigmoid(gi[:, d:2 * d] + gh[:, d:2 * d])
        nn = jnp.tanh(gi[:, 2 * d:] + r * gh[:, 2 * d:])
        hn = (1.0 - z) * nn + z * hh
        h_out[...] = hn
        m_out[...] = hn if first else jnp.maximum(m[...], hn)
        if not last:
            s_out[...] = dinv * (jnp.dot(hn, wg[...],
                                         preferred_element_type=jnp.float32)
                                 + bg[...])

    row_blk = pl.BlockSpec((_R, d), lambda i: (i, 0))
    deg_blk = pl.BlockSpec((_R, 16), lambda i: (i, 0))
    w3_blk = pl.BlockSpec((d, 3 * d), lambda i: (0, 0))
    b3_blk = pl.BlockSpec((1, 3 * d), lambda i: (0, 0))
    w_blk = pl.BlockSpec((d, d), lambda i: (0, 0))
    b_blk = pl.BlockSpec((1, d), lambda i: (0, 0))

    in_specs = [row_blk, row_blk, row_blk, row_blk]
    if not first:
        in_specs.append(row_blk)
    in_specs += [deg_blk, deg_blk, w3_blk, w3_blk, b3_blk, b3_blk]
    if not last:
        in_specs += [w_blk, b_blk]
    n_out = 2 if last else 3
    return pl.pallas_call(
        body,
        grid=(n_pad // _R,),
        in_specs=in_specs,
        out_specs=[row_blk] * n_out,
        out_shape=[jax.ShapeDtypeStruct((n_pad, d), jnp.float32)] * n_out,
    )


def kernel(x, edge_index, Wfc, bfc, Wg1, bg1, Wg2, bg2, Wg3, bg3,
           Wih1, Whh1, bih1, bhh1, Wih2, Whh2, bih2, bhh2,
           Wih3, Whh3, bih3, bhh3):
    n = x.shape[0]
    f_in = x.shape[1]
    d = Wfc.shape[0]
    e = edge_index.shape[1]

    n_pad = -(-(n + 1) // _R) * _R
    e_pad = -(-e // (_NW * _CH)) * (_NW * _CH)
    cpt = e_pad // (_NW * _CH)
    rpt = n_pad // _NS

    row = edge_index[0]
    col = edge_index[1]
    fill = jnp.full((e_pad - e,), n, dtype=jnp.int32)
    row3 = jnp.concatenate([row, fill]).reshape(_NW, cpt, _CH)
    col3 = jnp.concatenate([col, fill]).reshape(_NW, cpt, _CH)

    ones16 = jnp.ones((_CH, 16), jnp.float32)
    z16 = jnp.zeros((rpt, 16), jnp.float32)
    zd = jnp.zeros((rpt, d), jnp.float32)

    degp = _make_deg_kernel(n_pad, cpt)(row3, ones16, z16)
    d0, d1 = degp[0], degp[1]

    xp = jnp.zeros((n_pad, d), jnp.float32).at[:n, :f_in].set(x)
    wfcT = jnp.zeros((d, d), jnp.float32).at[:f_in, :].set(Wfc.T)
    h0, s1 = _make_prep_kernel(n_pad, d)(
        xp, wfcT, bfc[None], d0, d1, Wg1.T, bg1[None])

    spmm = _make_spmm_kernel(n_pad, cpt, d)
    layer1 = _make_layer_kernel(n_pad, d, first=True, last=False)
    layer2 = _make_layer_kernel(n_pad, d, first=False, last=False)
    layer3 = _make_layer_kernel(n_pad, d, first=False, last=True)

    p = spmm(s1, row3, col3, zd)
    h1, m1, s2 = layer1(p[0], p[1], s1, h0, d0, d1,
                        Wih1.T, Whh1.T, bih1[None], bhh1[None],
                        Wg2.T, bg2[None])
    p = spmm(s2, row3, col3, zd)
    h2, m2, s3 = layer2(p[0], p[1], s2, h1, m1, d0, d1,
                        Wih2.T, Whh2.T, bih2[None], bhh2[None],
                        Wg3.T, bg3[None])
    p = spmm(s3, row3, col3, zd)
    h3, m3 = layer3(p[0], p[1], s3, h2, m2, d0, d1,
                    Wih3.T, Whh3.T, bih3[None], bhh3[None])
    return m3[:n]
